# Initial kernel scaffold; baseline (speedup 1.0000x reference)
import jax
import jax.numpy as jnp
from jax.experimental import pallas as pl


def kernel(users, items, user_emb, item_emb, other_emb, edge_index, edge_vals):
    raise NotImplementedError("write your pallas kernel here")


# speedup vs baseline: 4.2134x; 4.2134x over previous
import functools

import jax
import jax.numpy as jnp
from jax import lax
from jax.experimental import pallas as pl
from jax.experimental.pallas import tpu as pltpu
from jax.experimental.pallas import tpu_sc as plsc

_NUM_USERS = 5000
_NUM_ITEMS = 4000
_NUM_OTHERS = 1000
_N_NODES = _NUM_USERS + _NUM_ITEMS + _NUM_OTHERS
_N_PAD = 10240
_DIM = 128
_N_LAYERS = 3
_K = 128
_L = 16


@functools.lru_cache(maxsize=None)
def _make_propagate(nc: int, ns: int, n_chunks: int):
    nrows_tile = _N_PAD // ns
    assert _N_PAD % ns == 0 and nrows_tile % _K == 0
    mesh = plsc.VectorSubcoreMesh(
        core_axis_name="c", subcore_axis_name="s",
        num_cores=nc, num_subcores=ns)

    @functools.partial(
        pl.kernel,
        out_type=jax.ShapeDtypeStruct((nc, _N_PAD, _DIM), jnp.float32),
        mesh=mesh,
        scratch_types=[
            pltpu.VMEM((n_chunks, _K), jnp.int32),
            pltpu.VMEM((n_chunks, _K), jnp.int32),
            pltpu.VMEM((n_chunks * _K,), jnp.float32),
            pltpu.VMEM((_K, _DIM), jnp.float32),
            pltpu.VMEM_SHARED((_N_PAD, _DIM), jnp.float32),
        ],
    )
    def propagate(emb, srcs, dsts, vals, out, src_v, dst_v, val_v, rows_v, acc):
        c = lax.axis_index("c")
        s = lax.axis_index("s")
        wid = c * ns + s

        zero = jnp.zeros((_L,), jnp.float32)

        def zrow(r, carry):
            for d in range(_DIM // _L):
                rows_v[r, pl.ds(d * _L, _L)] = zero
            return carry

        lax.fori_loop(0, _K, zrow, 0)
        base = s * nrows_tile
        nzc = nrows_tile // _K
        for j in range(nzc):
            pltpu.sync_copy(rows_v, acc.at[pl.ds(base + j * _K, _K)])
        plsc.subcore_barrier()

        pltpu.sync_copy(srcs.at[wid], src_v)
        pltpu.sync_copy(dsts.at[wid], dst_v)
        pltpu.sync_copy(vals.at[pl.ds(wid * n_chunks * _K, n_chunks * _K)],
                        val_v)

        def chunk(i, carry):
            pltpu.sync_copy(emb.at[src_v.at[i]], rows_v)

            def group(g, inner):
                vv = val_v[pl.ds(i * _K + g * _L, _L)]
                for j in range(_L):
                    e = g * _L + j
                    vb = jnp.full((_L,), vv[j], jnp.float32)
                    for d in range(_DIM // _L):
                        sl = pl.ds(d * _L, _L)
                        rows_v[e, sl] = rows_v[e, sl] * vb
                return inner

            lax.fori_loop(0, _K // _L, group, 0)
            pltpu.sync_copy(rows_v, acc.at[dst_v.at[i]], add=True)
            return carry

        lax.fori_loop(0, n_chunks, chunk, 0)
        plsc.subcore_barrier()

        for j in range(nzc):
            sl = pl.ds(base + j * _K, _K)
            pltpu.sync_copy(acc.at[sl], out.at[c].at[sl])

    return propagate


@functools.lru_cache(maxsize=None)
def _make_readout(nc: int, ns: int, batch: int):
    bt = batch // (nc * ns)
    mesh = plsc.VectorSubcoreMesh(
        core_axis_name="c", subcore_axis_name="s",
        num_cores=nc, num_subcores=ns)

    @functools.partial(
        pl.kernel,
        out_type=jax.ShapeDtypeStruct((batch,), jnp.float32),
        mesh=mesh,
        scratch_types=[
            pltpu.VMEM((bt,), jnp.int32),
            pltpu.VMEM((bt,), jnp.int32),
            pltpu.VMEM((bt, _DIM), jnp.float32),
            pltpu.VMEM((bt, _DIM), jnp.float32),
            pltpu.VMEM((bt,), jnp.float32),
        ],
    )
    def readout(emb, users, items, out, uidx, iidx, urows, irows, gam):
        c = lax.axis_index("c")
        s = lax.axis_index("s")
        wid = c * ns + s
        base = wid * bt
        pltpu.sync_copy(users.at[pl.ds(base, bt)], uidx)
        pltpu.sync_copy(items.at[pl.ds(base, bt)], iidx)
        pltpu.sync_copy(emb.at[uidx], urows)
        pltpu.sync_copy(emb.at[iidx], irows)

        lanes = lax.iota(jnp.int32, _L)

        def group(g, carry):
            res = jnp.zeros((_L,), jnp.float32)
            for j in range(_L):
                e = g * _L + j
                acc = jnp.zeros((_L,), jnp.float32)
                for d in range(_DIM // _L):
                    sl = pl.ds(d * _L, _L)
                    acc = acc + urows[e, sl] * irows[e, sl]
                s = acc[0]
                for t in range(1, _L):
                    s = s + acc[t]
                res = jnp.where(lanes == j, jnp.full((_L,), s), res)
            gam[pl.ds(g * _L, _L)] = res * (1.0 / 16.0)
            return carry

        lax.fori_loop(0, bt // _L, group, 0)
        pltpu.sync_copy(gam, out.at[pl.ds(base, bt)])

    return readout


def kernel(users, items, user_emb, item_emb, other_emb, edge_index, edge_vals):
    info = plsc.get_sparse_core_info()
    nc, ns = info.num_cores, info.num_subcores
    nw = nc * ns

    emb0 = jnp.concatenate([user_emb, item_emb, other_emb], axis=0)
    emb0 = jnp.pad(emb0, ((0, _N_PAD - _N_NODES), (0, 0)))
    n_edges = edge_vals.shape[0]
    n_chunks = -(-n_edges // (nw * _K))
    e_pad = nw * n_chunks * _K
    pad = e_pad - n_edges
    srcs = jnp.pad(edge_index[0], (0, pad)).reshape(nw, n_chunks, _K)
    dsts = jnp.pad(edge_index[1], (0, pad)).reshape(nw, n_chunks, _K)
    vals = jnp.pad(edge_vals, (0, pad))

    propagate = _make_propagate(nc, ns, n_chunks)
    emb = emb0
    total = emb0
    for _ in range(_N_LAYERS):
        partials = propagate(emb, srcs, dsts, vals)
        emb = jnp.sum(partials, axis=0)
        total = total + emb

    batch = users.shape[0]
    readout = _make_readout(nc, ns, batch)
    return readout(total, users, items + _NUM_USERS)
